# baseline (speedup 1.0000x reference)
import numpy as np
import jax
import jax.numpy as jnp
from jax import lax
from jax.experimental import pallas as pl
from jax.experimental.pallas import tpu as pltpu
from jax.experimental.pallas import tpu_sc as plsc

B = 4
S = 8192
D = 1024
PAD = 0

NC = 2
NS = 16
NW = NC * NS
POS_PER_W = S // NW
P = 32
N_CHUNKS = POS_PER_W // P
GROUPS = D // 16


def _pos_encoding(seq_len, d_model):
    pos = np.arange(seq_len, dtype=np.float32)[:, None]
    i = np.arange(0, d_model, 2, dtype=np.float32)
    div = np.exp(-np.log(10000.0) * i / float(d_model))
    pe = np.zeros((seq_len, d_model), dtype=np.float32)
    pe[:, 0::2] = np.sin(pos * div)
    pe[:, 1::2] = np.cos(pos * div)
    return jnp.asarray(pe)


def _body(x_hbm, pe_hbm, tbl_hbm, out_hbm, idx_v, pe_v, rows_v, sem):
    wid = lax.axis_index("s") * NC + lax.axis_index("c")
    pos0 = wid * POS_PER_W

    def chunk_body(c, _):
        pstart = pos0 + c * P
        pltpu.sync_copy(pe_hbm.at[pl.ds(pstart, P)], pe_v)

        def batch_body(b, _):
            pltpu.sync_copy(x_hbm.at[b, pl.ds(pstart, P)], idx_v.at[pl.ds(0, P)])
            pltpu.async_copy(tbl_hbm.at[idx_v.at[pl.ds(0, P)]], rows_v, sem).wait()

            def row_body(r, _):
                ivec = idx_v[pl.ds(r, 16)]
                scale = jnp.where(ivec[0] != PAD, 1.0, 0.0).astype(jnp.float32)

                def col_body(g, _):
                    off = g * 16
                    row = rows_v[r, pl.ds(off, 16)]
                    pev = pe_v[r, pl.ds(off, 16)]
                    rows_v[r, pl.ds(off, 16)] = row * scale + pev
                    return 0

                lax.fori_loop(0, GROUPS, col_body, 0)
                return 0

            lax.fori_loop(0, P, row_body, 0)
            pltpu.sync_copy(rows_v, out_hbm.at[b, pl.ds(pstart, P)])
            return 0

        lax.fori_loop(0, B, batch_body, 0)
        return 0

    lax.fori_loop(0, N_CHUNKS, chunk_body, 0)


def kernel(x, token_emb_weight):
    pe = _pos_encoding(S, D)
    mesh = plsc.VectorSubcoreMesh(core_axis_name="c", subcore_axis_name="s")
    k = pl.kernel(
        _body,
        out_type=jax.ShapeDtypeStruct((B, S, D), jnp.float32),
        mesh=mesh,
        scratch_types=[
            pltpu.VMEM((P + 16,), jnp.int32),
            pltpu.VMEM((P, D), jnp.float32),
            pltpu.VMEM((P, D), jnp.float32),
            pltpu.SemaphoreType.DMA,
        ],
    )
    return k(x, pe, token_emb_weight)


# speedup vs baseline: 2.1309x; 2.1309x over previous
import numpy as np
import jax
import jax.numpy as jnp
from jax import lax
from jax.experimental import pallas as pl
from jax.experimental.pallas import tpu as pltpu
from jax.experimental.pallas import tpu_sc as plsc

B = 4
S = 8192
D = 1024
PAD = 0

NC = 2
NS = 16
NW = NC * NS
POS_PER_W = S // NW
P = 32
N_CHUNKS = POS_PER_W // P
GROUPS = D // 16


def _pos_encoding(seq_len, d_model):
    pos = np.arange(seq_len, dtype=np.float32)[:, None]
    i = np.arange(0, d_model, 2, dtype=np.float32)
    div = np.exp(-np.log(10000.0) * i / float(d_model))
    pe = np.zeros((seq_len, d_model), dtype=np.float32)
    pe[:, 0::2] = np.sin(pos * div)
    pe[:, 1::2] = np.cos(pos * div)
    return jnp.asarray(pe)


def _body(x_hbm, pe_hbm, tbl_hbm, out_hbm, idx_v, pe_v, rows_v, sem):
    wid = lax.axis_index("s") * NC + lax.axis_index("c")
    pos0 = wid * POS_PER_W

    def chunk_body(c, _):
        pstart = pos0 + c * P
        pltpu.sync_copy(pe_hbm.at[pl.ds(pstart, P)], pe_v)

        def batch_body(b, _):
            pltpu.sync_copy(x_hbm.at[b, pl.ds(pstart, P)], idx_v.at[pl.ds(0, P)])
            pltpu.async_copy(tbl_hbm.at[idx_v.at[pl.ds(0, P)]], rows_v, sem).wait()

            def row_body(r, _):
                ivec = idx_v[pl.ds(r, 16)]
                scale = jnp.where(ivec[0] != PAD, 1.0, 0.0).astype(jnp.float32)

                @plsc.parallel_loop(0, D, 16, unroll=8)
                def _add(off):
                    rows_v[r, pl.ds(off, 16)] = (
                        rows_v[r, pl.ds(off, 16)] * scale
                        + pe_v[r, pl.ds(off, 16)]
                    )
                return 0

            lax.fori_loop(0, P, row_body, 0)

            pltpu.sync_copy(rows_v, out_hbm.at[b, pl.ds(pstart, P)])
            return 0

        lax.fori_loop(0, B, batch_body, 0)
        return 0

    lax.fori_loop(0, N_CHUNKS, chunk_body, 0)


def kernel(x, token_emb_weight):
    pe = _pos_encoding(S, D)
    mesh = plsc.VectorSubcoreMesh(core_axis_name="c", subcore_axis_name="s")
    k = pl.kernel(
        _body,
        out_type=jax.ShapeDtypeStruct((B, S, D), jnp.float32),
        mesh=mesh,
        scratch_types=[
            pltpu.VMEM((P + 16,), jnp.int32),
            pltpu.VMEM((P, D), jnp.float32),
            pltpu.VMEM((P, D), jnp.float32),
            pltpu.SemaphoreType.DMA,
        ],
    )
    return k(x, pe, token_emb_weight)


# speedup vs baseline: 3.3108x; 1.5537x over previous
import numpy as np
import jax
import jax.numpy as jnp
from jax import lax
from jax.experimental import pallas as pl
from jax.experimental.pallas import tpu as pltpu
from jax.experimental.pallas import tpu_sc as plsc

B = 4
S = 8192
D = 1024
PAD = 0

NC = 2
NS = 16
NW = NC * NS
POS_PER_W = S // NW
P = 16
N_CHUNKS = POS_PER_W // P


def _pos_encoding(seq_len, d_model):
    pos = np.arange(seq_len, dtype=np.float32)[:, None]
    i = np.arange(0, d_model, 2, dtype=np.float32)
    div = np.exp(-np.log(10000.0) * i / float(d_model))
    pe = np.zeros((seq_len, d_model), dtype=np.float32)
    pe[:, 0::2] = np.sin(pos * div)
    pe[:, 1::2] = np.cos(pos * div)
    return jnp.asarray(pe)


def _body(x_hbm, pe_hbm, tbl_hbm, out_hbm,
          idx_all, scale_v, pe0, pe1, rb0, rb1, g0, g1, s0, s1, q0, q1):
    wid = lax.axis_index("s") * NC + lax.axis_index("c")
    pos0 = wid * POS_PER_W
    peb = (pe0, pe1)
    rbb = (rb0, rb1)
    gs = (g0, g1)
    ss = (s0, s1)
    qs = (q0, q1)

    pltpu.sync_copy(x_hbm.at[:, pl.ds(pos0, POS_PER_W)],
                    idx_all.at[:, pl.ds(0, POS_PER_W)])
    pltpu.async_copy(pe_hbm.at[pl.ds(pos0, P)], pe0, q0)
    pltpu.async_copy(tbl_hbm.at[idx_all.at[0, pl.ds(0, P)]], rb0, g0)

    def cpair_body(cp, _):
        for cc in range(2):
            c = cp * 2 + cc
            coff = c * P
            pstart = pos0 + coff
            pltpu.make_async_copy(
                pe_hbm.at[pl.ds(pstart, P)], peb[cc], qs[cc]).wait()

            @pl.when(c + 1 < N_CHUNKS)
            def _():
                pltpu.async_copy(
                    pe_hbm.at[pl.ds(pstart + P, P)], peb[1 - cc], qs[1 - cc])

            for b in range(B):
                j = b % 2
                nj = 1 - j
                if b < B - 1:
                    if b == 0:
                        @pl.when(c > 0)
                        def _():
                            pltpu.make_async_copy(
                                rbb[nj],
                                out_hbm.at[B - 1, pl.ds(pstart - P, P)],
                                ss[nj]).wait()
                    else:
                        pltpu.make_async_copy(
                            rbb[nj], out_hbm.at[b - 1, pl.ds(pstart, P)],
                            ss[nj]).wait()
                    pltpu.async_copy(
                        tbl_hbm.at[idx_all.at[b + 1, pl.ds(coff, P)]],
                        rbb[nj], gs[nj])
                else:
                    @pl.when(c + 1 < N_CHUNKS)
                    def _():
                        pltpu.make_async_copy(
                            rbb[0], out_hbm.at[B - 2, pl.ds(pstart, P)],
                            ss[0]).wait()
                        pltpu.async_copy(
                            tbl_hbm.at[idx_all.at[0, pl.ds(coff + P, P)]],
                            rbb[0], gs[0])
                pltpu.make_async_copy(
                    tbl_hbm.at[idx_all.at[b, pl.ds(coff, P)]],
                    rbb[j], gs[j]).wait()

                ivec = idx_all[b, pl.ds(coff, 16)]
                scale_v[pl.ds(0, 16)] = jnp.where(
                    ivec != PAD, 1.0, 0.0).astype(jnp.float32)

                def row_body(r, _):
                    svec = scale_v[pl.ds(r, 16)]
                    scale = svec[0]

                    @plsc.parallel_loop(0, D, 16, unroll=8)
                    def _add(off):
                        rbb[j][r, pl.ds(off, 16)] = (
                            rbb[j][r, pl.ds(off, 16)] * scale
                            + peb[cc][r, pl.ds(off, 16)]
                        )
                    return 0

                lax.fori_loop(0, P, row_body, 0)
                pltpu.async_copy(
                    rbb[j], out_hbm.at[b, pl.ds(pstart, P)], ss[j])
        return 0

    lax.fori_loop(0, N_CHUNKS // 2, cpair_body, 0)
    last = pos0 + (N_CHUNKS - 1) * P
    pltpu.make_async_copy(rb0, out_hbm.at[B - 2, pl.ds(last, P)], s0).wait()
    pltpu.make_async_copy(rb1, out_hbm.at[B - 1, pl.ds(last, P)], s1).wait()


def kernel(x, token_emb_weight):
    pe = _pos_encoding(S, D)
    mesh = plsc.VectorSubcoreMesh(core_axis_name="c", subcore_axis_name="s")
    k = pl.kernel(
        _body,
        out_type=jax.ShapeDtypeStruct((B, S, D), jnp.float32),
        mesh=mesh,
        scratch_types=[
            pltpu.VMEM((B, POS_PER_W + 16), jnp.int32),
            pltpu.VMEM((P + 16,), jnp.float32),
            pltpu.VMEM((P, D), jnp.float32),
            pltpu.VMEM((P, D), jnp.float32),
            pltpu.VMEM((P, D), jnp.float32),
            pltpu.VMEM((P, D), jnp.float32),
            pltpu.SemaphoreType.DMA,
            pltpu.SemaphoreType.DMA,
            pltpu.SemaphoreType.DMA,
            pltpu.SemaphoreType.DMA,
            pltpu.SemaphoreType.DMA,
            pltpu.SemaphoreType.DMA,
        ],
    )
    return k(x, pe, token_emb_weight)


# speedup vs baseline: 3.8214x; 1.1542x over previous
import numpy as np
import jax
import jax.numpy as jnp
from jax import lax
from jax.experimental import pallas as pl
from jax.experimental.pallas import tpu as pltpu
from jax.experimental.pallas import tpu_sc as plsc

B = 4
S = 8192
D = 1024
PAD = 0

NC = 2
NS = 16
NW = NC * NS
POS_PER_W = S // NW
P = 16
N_CHUNKS = POS_PER_W // P
SSTR = P + 16


def _pos_encoding(seq_len, d_model):
    pos = np.arange(seq_len, dtype=np.float32)[:, None]
    i = np.arange(0, d_model, 2, dtype=np.float32)
    div = np.exp(-np.log(10000.0) * i / float(d_model))
    pe = np.zeros((seq_len, d_model), dtype=np.float32)
    pe[:, 0::2] = np.sin(pos * div)
    pe[:, 1::2] = np.cos(pos * div)
    return jnp.asarray(pe)


def _body(x_hbm, pe_hbm, tbl_hbm, out_hbm,
          idx_all, scale_v, pe0, pe1, rb0, rb1, g0, g1, s0, s1, q0, q1):
    wid = lax.axis_index("s") * NC + lax.axis_index("c")
    pos0 = wid * POS_PER_W
    peb = (pe0, pe1)
    rbb = (rb0, rb1)
    gs = (g0, g1)
    ss = (s0, s1)
    qs = (q0, q1)

    pltpu.sync_copy(x_hbm.at[:, pl.ds(pos0, POS_PER_W)],
                    idx_all.at[:, pl.ds(0, POS_PER_W)])
    pltpu.async_copy(pe_hbm.at[pl.ds(pos0, P)], pe0, q0)
    pltpu.async_copy(tbl_hbm.at[idx_all.at[0, pl.ds(0, P)]], rb0.at[0], g0)
    pltpu.async_copy(tbl_hbm.at[idx_all.at[1, pl.ds(0, P)]], rb0.at[1], g0)

    def cpair_body(cp, _):
        for cc in range(2):
            c = cp * 2 + cc
            coff = c * P
            pstart = pos0 + coff
            pltpu.make_async_copy(
                pe_hbm.at[pl.ds(pstart, P)], peb[cc], qs[cc]).wait()

            @pl.when(c + 1 < N_CHUNKS)
            def _():
                pltpu.async_copy(
                    pe_hbm.at[pl.ds(pstart + P, P)], peb[1 - cc], qs[1 - cc])

            for h in range(2):
                j = h
                nj = 1 - j
                if h == 0:
                    @pl.when(c > 0)
                    def _():
                        pltpu.make_async_copy(
                            rbb[nj].at[0],
                            out_hbm.at[2, pl.ds(pstart - P, P)],
                            ss[nj]).wait()
                        pltpu.make_async_copy(
                            rbb[nj].at[1],
                            out_hbm.at[3, pl.ds(pstart - P, P)],
                            ss[nj]).wait()
                    pltpu.async_copy(
                        tbl_hbm.at[idx_all.at[2, pl.ds(coff, P)]],
                        rbb[nj].at[0], gs[nj])
                    pltpu.async_copy(
                        tbl_hbm.at[idx_all.at[3, pl.ds(coff, P)]],
                        rbb[nj].at[1], gs[nj])
                else:
                    @pl.when(c + 1 < N_CHUNKS)
                    def _():
                        pltpu.make_async_copy(
                            rbb[0].at[0], out_hbm.at[0, pl.ds(pstart, P)],
                            ss[0]).wait()
                        pltpu.make_async_copy(
                            rbb[0].at[1], out_hbm.at[1, pl.ds(pstart, P)],
                            ss[0]).wait()
                        pltpu.async_copy(
                            tbl_hbm.at[idx_all.at[0, pl.ds(coff + P, P)]],
                            rbb[0].at[0], gs[0])
                        pltpu.async_copy(
                            tbl_hbm.at[idx_all.at[1, pl.ds(coff + P, P)]],
                            rbb[0].at[1], gs[0])
                pltpu.make_async_copy(
                    tbl_hbm.at[idx_all.at[2 * h, pl.ds(coff, P)]],
                    rbb[j].at[0], gs[j]).wait()
                pltpu.make_async_copy(
                    tbl_hbm.at[idx_all.at[2 * h + 1, pl.ds(coff, P)]],
                    rbb[j].at[1], gs[j]).wait()

                for bl in range(2):
                    ivec = idx_all[2 * h + bl, pl.ds(coff, 16)]
                    scale_v[pl.ds(bl * SSTR, 16)] = jnp.where(
                        ivec != PAD, 1.0, 0.0).astype(jnp.float32)

                def row_body(r, _):
                    sc0 = scale_v[pl.ds(r, 16)][0]
                    sc1 = scale_v[pl.ds(SSTR + r, 16)][0]

                    @plsc.parallel_loop(0, D, 16, unroll=4)
                    def _add(off):
                        peg = peb[cc][r, pl.ds(off, 16)]
                        rbb[j][0, r, pl.ds(off, 16)] = (
                            rbb[j][0, r, pl.ds(off, 16)] * sc0 + peg)
                        rbb[j][1, r, pl.ds(off, 16)] = (
                            rbb[j][1, r, pl.ds(off, 16)] * sc1 + peg)
                    return 0

                lax.fori_loop(0, P, row_body, 0)
                pltpu.async_copy(
                    rbb[j].at[0], out_hbm.at[2 * h, pl.ds(pstart, P)], ss[j])
                pltpu.async_copy(
                    rbb[j].at[1], out_hbm.at[2 * h + 1, pl.ds(pstart, P)],
                    ss[j])
        return 0

    lax.fori_loop(0, N_CHUNKS // 2, cpair_body, 0)
    last = pos0 + (N_CHUNKS - 1) * P
    pltpu.make_async_copy(rb0.at[0], out_hbm.at[0, pl.ds(last, P)], s0).wait()
    pltpu.make_async_copy(rb0.at[1], out_hbm.at[1, pl.ds(last, P)], s0).wait()
    pltpu.make_async_copy(rb1.at[0], out_hbm.at[2, pl.ds(last, P)], s1).wait()
    pltpu.make_async_copy(rb1.at[1], out_hbm.at[3, pl.ds(last, P)], s1).wait()


def kernel(x, token_emb_weight):
    pe = _pos_encoding(S, D)
    mesh = plsc.VectorSubcoreMesh(core_axis_name="c", subcore_axis_name="s")
    k = pl.kernel(
        _body,
        out_type=jax.ShapeDtypeStruct((B, S, D), jnp.float32),
        mesh=mesh,
        scratch_types=[
            pltpu.VMEM((B, POS_PER_W + 16), jnp.int32),
            pltpu.VMEM((2 * SSTR,), jnp.float32),
            pltpu.VMEM((P, D), jnp.float32),
            pltpu.VMEM((P, D), jnp.float32),
            pltpu.VMEM((2, P, D), jnp.float32),
            pltpu.VMEM((2, P, D), jnp.float32),
            pltpu.SemaphoreType.DMA,
            pltpu.SemaphoreType.DMA,
            pltpu.SemaphoreType.DMA,
            pltpu.SemaphoreType.DMA,
            pltpu.SemaphoreType.DMA,
            pltpu.SemaphoreType.DMA,
        ],
    )
    return k(x, pe, token_emb_weight)
